# Initial kernel scaffold; baseline (speedup 1.0000x reference)
import jax
import jax.numpy as jnp
from jax.experimental import pallas as pl


def kernel(features, edge_index, W1_self, W1_neigh, b1, W2_self, W2_neigh, b2):
    raise NotImplementedError("write your pallas kernel here")


# speedup vs baseline: 5.8257x; 5.8257x over previous
import functools

import jax
import jax.numpy as jnp
from jax import lax
from jax.experimental import pallas as pl
from jax.experimental.pallas import tpu as pltpu
from jax.experimental.pallas import tpu_sc as plsc

N = 10000
E = 320000
DIN = 128
DHID = 256
DOUT = 64

NC = 2
NS = 16
NW = NC * NS
B = 128
NB = 79
EPW = NB * B
EPAD = EPW * NW
NP = 10240
RPT = NP // NS
DEGW = 8


def _seg_sum_kernel(D, with_deg):
    mesh = plsc.VectorSubcoreMesh(core_axis_name="c", subcore_axis_name="s")
    out_type = [jax.ShapeDtypeStruct((NC, NP, D), jnp.float32)]
    scratch = [
        pltpu.VMEM_SHARED((NP, D), jnp.float32),
        pltpu.VMEM((B,), jnp.int32),
        pltpu.VMEM((B,), jnp.int32),
        pltpu.VMEM((B, D), jnp.float32),
        pltpu.SemaphoreType.DMA,
    ]
    if with_deg:
        out_type.append(jax.ShapeDtypeStruct((NC, NP, DEGW), jnp.float32))
        scratch += [
            pltpu.VMEM_SHARED((NP, DEGW), jnp.float32),
            pltpu.VMEM((B, DEGW), jnp.float32),
            pltpu.VMEM((B, DEGW), jnp.float32),
        ]

    def body(*refs):
        if with_deg:
            (table_hbm, src_hbm, dst_hbm, zer_hbm, one_hbm, z8_hbm,
             agg_out, deg_out,
             agg_sh, idx_s, idx_d, rows, sem, deg_sh, one, b8) = refs
        else:
            (table_hbm, src_hbm, dst_hbm, zer_hbm,
             agg_out,
             agg_sh, idx_s, idx_d, rows, sem) = refs
        c = lax.axis_index("c")
        s = lax.axis_index("s")
        wid = c * NS + s
        r0 = s * RPT

        pltpu.sync_copy(zer_hbm, rows)
        if with_deg:
            pltpu.sync_copy(one_hbm, one)
            pltpu.sync_copy(z8_hbm, b8)
        for k in range(RPT // B):
            pltpu.sync_copy(rows, agg_sh.at[pl.ds(r0 + k * B, B)])
            if with_deg:
                pltpu.sync_copy(b8, deg_sh.at[pl.ds(r0 + k * B, B)])
        plsc.subcore_barrier()

        def step(j, carry):
            off = pl.multiple_of(wid * EPW + j * B, B)
            pltpu.sync_copy(src_hbm.at[pl.ds(off, B)], idx_s)
            pltpu.sync_copy(dst_hbm.at[pl.ds(off, B)], idx_d)
            pltpu.async_copy(table_hbm.at[idx_s], rows, sem).wait()
            pltpu.sync_copy(rows, agg_sh.at[idx_d], add=True)
            if with_deg:
                pltpu.sync_copy(one, deg_sh.at[idx_d], add=True)
            return carry

        lax.fori_loop(0, NB, step, 0)
        plsc.subcore_barrier()

        for k in range(RPT // B):
            r = r0 + k * B
            pltpu.sync_copy(agg_sh.at[pl.ds(r, B)], rows)
            pltpu.sync_copy(rows, agg_out.at[c, pl.ds(r, B)])
            if with_deg:
                pltpu.sync_copy(deg_sh.at[pl.ds(r, B)], b8)
                pltpu.sync_copy(b8, deg_out.at[c, pl.ds(r, B)])

    return pl.kernel(body, out_type=tuple(out_type), mesh=mesh,
                     scratch_types=tuple(scratch),
                     compiler_params=pltpu.CompilerParams(
                         use_tc_tiling_on_sc=False))


_seg128 = _seg_sum_kernel(DIN, with_deg=True)
_seg64 = _seg_sum_kernel(DOUT, with_deg=False)


def _tc1_body(x_ref, agg_ref, deg_ref, w1s_ref, w1n_ref, b1_ref,
              w2s_ref, w2n_ref, b2_ref, z_ref, s2_ref):
    x = x_ref[...]
    agg = agg_ref[0] + agg_ref[1]
    dg = deg_ref[0, :, 0:1] + deg_ref[1, :, 0:1]
    inv = 1.0 / jnp.maximum(dg, 1.0)
    hn = agg * inv
    h = jnp.dot(x, w1s_ref[...], preferred_element_type=jnp.float32)
    h = h + jnp.dot(hn, w1n_ref[...], preferred_element_type=jnp.float32)
    h = h + b1_ref[...]
    z_ref[...] = jnp.dot(h, w2n_ref[...], preferred_element_type=jnp.float32)
    s2_ref[...] = (jnp.dot(h, w2s_ref[...], preferred_element_type=jnp.float32)
                   + b2_ref[...])


def _tc2_body(s2_ref, agg2_ref, deg_ref, o_ref):
    dg = deg_ref[0, :, 0:1] + deg_ref[1, :, 0:1]
    inv = 1.0 / jnp.maximum(dg, 1.0)
    o_ref[...] = s2_ref[...] + (agg2_ref[0] + agg2_ref[1]) * inv


_R = 1000


def _tc1(x, agg1, deg, w1sT, w1nT, b1r, w2sT, w2nT, b2r):
    grid = (N // _R,)
    return pl.pallas_call(
        _tc1_body,
        grid=grid,
        in_specs=[
            pl.BlockSpec((_R, DIN), lambda i: (i, 0)),
            pl.BlockSpec((NC, _R, DIN), lambda i: (0, i, 0)),
            pl.BlockSpec((NC, _R, DEGW), lambda i: (0, i, 0)),
            pl.BlockSpec((DIN, DHID), lambda i: (0, 0)),
            pl.BlockSpec((DIN, DHID), lambda i: (0, 0)),
            pl.BlockSpec((1, DHID), lambda i: (0, 0)),
            pl.BlockSpec((DHID, DOUT), lambda i: (0, 0)),
            pl.BlockSpec((DHID, DOUT), lambda i: (0, 0)),
            pl.BlockSpec((1, DOUT), lambda i: (0, 0)),
        ],
        out_specs=[
            pl.BlockSpec((_R, DOUT), lambda i: (i, 0)),
            pl.BlockSpec((_R, DOUT), lambda i: (i, 0)),
        ],
        out_shape=[
            jax.ShapeDtypeStruct((N, DOUT), jnp.float32),
            jax.ShapeDtypeStruct((N, DOUT), jnp.float32),
        ],
    )(x, agg1, deg, w1sT, w1nT, b1r, w2sT, w2nT, b2r)


def _tc2(s2, agg2, deg):
    grid = (N // _R,)
    return pl.pallas_call(
        _tc2_body,
        grid=grid,
        in_specs=[
            pl.BlockSpec((_R, DOUT), lambda i: (i, 0)),
            pl.BlockSpec((NC, _R, DOUT), lambda i: (0, i, 0)),
            pl.BlockSpec((NC, _R, DEGW), lambda i: (0, i, 0)),
        ],
        out_specs=pl.BlockSpec((_R, DOUT), lambda i: (i, 0)),
        out_shape=jax.ShapeDtypeStruct((N, DOUT), jnp.float32),
    )(s2, agg2, deg)


def kernel(features, edge_index, W1_self, W1_neigh, b1, W2_self, W2_neigh, b2):
    src = edge_index[0].astype(jnp.int32)
    dst = edge_index[1].astype(jnp.int32)
    pad = EPAD - E
    src_p = jnp.concatenate([src, jnp.zeros((pad,), jnp.int32)])
    dst_p = jnp.concatenate([dst, jnp.full((pad,), N, jnp.int32)])

    zer128 = jnp.zeros((B, DIN), jnp.float32)
    zer64 = jnp.zeros((B, DOUT), jnp.float32)
    one8 = jnp.ones((B, DEGW), jnp.float32)
    zer8 = jnp.zeros((B, DEGW), jnp.float32)

    aggp, degp = _seg128(features, src_p, dst_p, zer128, one8, zer8)
    agg1 = aggp[:, :N]
    deg = degp[:, :N]

    z, s2 = _tc1(features, agg1, deg, W1_self.T, W1_neigh.T, b1[None],
                 W2_self.T, W2_neigh.T, b2[None])

    (agg2p,) = _seg64(z, src_p, dst_p, zer64)
    out = _tc2(s2, agg2p[:, :N], deg)
    return out


# speedup vs baseline: 6.5439x; 1.1233x over previous
import jax
import jax.numpy as jnp
from jax import lax
from jax.experimental import pallas as pl
from jax.experimental.pallas import tpu as pltpu
from jax.experimental.pallas import tpu_sc as plsc

N = 10000
E = 320000
DIN = 128
DHID = 256
DOUT = 64

NC = 2
NS = 16
NW = NC * NS
NP = 10016
RPT = NP // NS
DEGW = 8

B1, NB1 = 64, 158
B2, NB2 = 128, 80
EPAD1 = NW * NB1 * B1
EPAD2 = NW * NB2 * B2


def _seg_sum_kernel(D, B, NB, with_deg):
    assert NB % 2 == 0 and RPT > B
    mesh = plsc.VectorSubcoreMesh(core_axis_name="c", subcore_axis_name="s")
    out_type = [jax.ShapeDtypeStruct((NC, NP, D), jnp.float32)]
    scratch = [
        pltpu.VMEM_SHARED((NP, D), jnp.float32),
        pltpu.VMEM((NB, B), jnp.int32),
        pltpu.VMEM((NB, B), jnp.int32),
        pltpu.VMEM((B, D), jnp.float32),
        pltpu.VMEM((B, D), jnp.float32),
        pltpu.SemaphoreType.DMA,
        pltpu.SemaphoreType.DMA,
    ]
    if with_deg:
        out_type.append(jax.ShapeDtypeStruct((NC, NP, DEGW), jnp.float32))
        scratch += [
            pltpu.VMEM_SHARED((NP, DEGW), jnp.float32),
            pltpu.VMEM((B, DEGW), jnp.float32),
            pltpu.VMEM((B, DEGW), jnp.float32),
        ]
    nfull, rem = divmod(RPT, B)

    def body(*refs):
        if with_deg:
            (table_hbm, src_hbm, dst_hbm, zer_hbm, one_hbm, z8_hbm,
             agg_out, deg_out,
             agg_sh, idx_s_all, idx_d_all, rows0, rows1, sem_g, sem_s,
             deg_sh, one, b8) = refs
        else:
            (table_hbm, src_hbm, dst_hbm, zer_hbm,
             agg_out,
             agg_sh, idx_s_all, idx_d_all, rows0, rows1, sem_g, sem_s) = refs
        c = lax.axis_index("c")
        s = lax.axis_index("s")
        wid = c * NS + s
        r0 = s * RPT

        pltpu.sync_copy(src_hbm.at[wid], idx_s_all)
        pltpu.sync_copy(dst_hbm.at[wid], idx_d_all)
        pltpu.sync_copy(zer_hbm, rows0)
        if with_deg:
            pltpu.sync_copy(one_hbm, one)
            pltpu.sync_copy(z8_hbm, b8)

        for k in range(nfull):
            pltpu.sync_copy(rows0, agg_sh.at[pl.ds(r0 + k * B, B)])
            if with_deg:
                pltpu.sync_copy(b8, deg_sh.at[pl.ds(r0 + k * B, B)])
        if rem:
            r = r0 + nfull * B
            pltpu.sync_copy(rows0.at[pl.ds(0, rem)], agg_sh.at[pl.ds(r, rem)])
            if with_deg:
                pltpu.sync_copy(b8.at[pl.ds(0, rem)], deg_sh.at[pl.ds(r, rem)])
        plsc.subcore_barrier()

        def g_issue(j, buf):
            pltpu.async_copy(table_hbm.at[idx_s_all.at[j]], buf, sem_g)

        def g_wait(j, buf):
            pltpu.make_async_copy(table_hbm.at[idx_s_all.at[j]], buf,
                                  sem_g).wait()

        def s_do(j, buf):
            if with_deg:
                d2 = pltpu.async_copy(one, deg_sh.at[idx_d_all.at[j]], sem_s,
                                      add=True)
            pltpu.sync_copy(buf, agg_sh.at[idx_d_all.at[j]], add=True)
            if with_deg:
                d2.wait()

        g_issue(0, rows0)

        def pair(jj, carry):
            j0 = jj * 2
            g_wait(j0, rows0)
            g_issue(j0 + 1, rows1)
            s_do(j0, rows0)
            g_wait(j0 + 1, rows1)
            g_issue(j0 + 2, rows0)
            s_do(j0 + 1, rows1)
            return carry

        lax.fori_loop(0, NB // 2 - 1, pair, 0)
        j0 = NB - 2
        g_wait(j0, rows0)
        g_issue(j0 + 1, rows1)
        s_do(j0, rows0)
        g_wait(j0 + 1, rows1)
        s_do(j0 + 1, rows1)
        plsc.subcore_barrier()

        for k in range(nfull):
            r = r0 + k * B
            pltpu.sync_copy(agg_sh.at[pl.ds(r, B)], rows0)
            pltpu.sync_copy(rows0, agg_out.at[c, pl.ds(r, B)])
            if with_deg:
                pltpu.sync_copy(deg_sh.at[pl.ds(r, B)], b8)
                pltpu.sync_copy(b8, deg_out.at[c, pl.ds(r, B)])
        if rem:
            r = r0 + nfull * B
            pltpu.sync_copy(agg_sh.at[pl.ds(r, rem)], rows0.at[pl.ds(0, rem)])
            pltpu.sync_copy(rows0.at[pl.ds(0, rem)],
                            agg_out.at[c, pl.ds(r, rem)])
            if with_deg:
                pltpu.sync_copy(deg_sh.at[pl.ds(r, rem)], b8.at[pl.ds(0, rem)])
                pltpu.sync_copy(b8.at[pl.ds(0, rem)],
                                deg_out.at[c, pl.ds(r, rem)])

    return pl.kernel(body, out_type=tuple(out_type), mesh=mesh,
                     scratch_types=tuple(scratch),
                     compiler_params=pltpu.CompilerParams(
                         use_tc_tiling_on_sc=False))


_seg128 = _seg_sum_kernel(DIN, B1, NB1, with_deg=True)
_seg64 = _seg_sum_kernel(DOUT, B2, NB2, with_deg=False)


def _tc1_body(x_ref, agg_ref, deg_ref, w1s_ref, w1n_ref, b1_ref,
              w2s_ref, w2n_ref, b2_ref, z_ref, s2_ref):
    x = x_ref[...]
    agg = agg_ref[0] + agg_ref[1]
    dg = deg_ref[0, :, 0:1] + deg_ref[1, :, 0:1]
    inv = 1.0 / jnp.maximum(dg, 1.0)
    hn = agg * inv
    h = jnp.dot(x, w1s_ref[...], preferred_element_type=jnp.float32)
    h = h + jnp.dot(hn, w1n_ref[...], preferred_element_type=jnp.float32)
    h = h + b1_ref[...]
    z_ref[...] = jnp.dot(h, w2n_ref[...], preferred_element_type=jnp.float32)
    s2_ref[...] = (jnp.dot(h, w2s_ref[...], preferred_element_type=jnp.float32)
                   + b2_ref[...])


def _tc2_body(s2_ref, agg2_ref, deg_ref, o_ref):
    dg = deg_ref[0, :, 0:1] + deg_ref[1, :, 0:1]
    inv = 1.0 / jnp.maximum(dg, 1.0)
    o_ref[...] = s2_ref[...] + (agg2_ref[0] + agg2_ref[1]) * inv


_R = 1000


def _tc1(x, agg1, deg, w1sT, w1nT, b1r, w2sT, w2nT, b2r):
    grid = (N // _R,)
    return pl.pallas_call(
        _tc1_body,
        grid=grid,
        in_specs=[
            pl.BlockSpec((_R, DIN), lambda i: (i, 0)),
            pl.BlockSpec((NC, _R, DIN), lambda i: (0, i, 0)),
            pl.BlockSpec((NC, _R, DEGW), lambda i: (0, i, 0)),
            pl.BlockSpec((DIN, DHID), lambda i: (0, 0)),
            pl.BlockSpec((DIN, DHID), lambda i: (0, 0)),
            pl.BlockSpec((1, DHID), lambda i: (0, 0)),
            pl.BlockSpec((DHID, DOUT), lambda i: (0, 0)),
            pl.BlockSpec((DHID, DOUT), lambda i: (0, 0)),
            pl.BlockSpec((1, DOUT), lambda i: (0, 0)),
        ],
        out_specs=[
            pl.BlockSpec((_R, DOUT), lambda i: (i, 0)),
            pl.BlockSpec((_R, DOUT), lambda i: (i, 0)),
        ],
        out_shape=[
            jax.ShapeDtypeStruct((N, DOUT), jnp.float32),
            jax.ShapeDtypeStruct((N, DOUT), jnp.float32),
        ],
    )(x, agg1, deg, w1sT, w1nT, b1r, w2sT, w2nT, b2r)


def _tc2(s2, agg2, deg):
    grid = (N // _R,)
    return pl.pallas_call(
        _tc2_body,
        grid=grid,
        in_specs=[
            pl.BlockSpec((_R, DOUT), lambda i: (i, 0)),
            pl.BlockSpec((NC, _R, DOUT), lambda i: (0, i, 0)),
            pl.BlockSpec((NC, _R, DEGW), lambda i: (0, i, 0)),
        ],
        out_specs=pl.BlockSpec((_R, DOUT), lambda i: (i, 0)),
        out_shape=jax.ShapeDtypeStruct((N, DOUT), jnp.float32),
    )(s2, agg2, deg)


def _pad_edges(src, dst, epad):
    pad = epad - E
    src_p = jnp.concatenate([src, jnp.zeros((pad,), jnp.int32)])
    dst_p = jnp.concatenate([dst, jnp.full((pad,), N, jnp.int32)])
    return src_p, dst_p


def kernel(features, edge_index, W1_self, W1_neigh, b1, W2_self, W2_neigh, b2):
    src = edge_index[0].astype(jnp.int32)
    dst = edge_index[1].astype(jnp.int32)
    s1, d1 = _pad_edges(src, dst, EPAD1)
    s1 = s1.reshape(NW, NB1, B1)
    d1 = d1.reshape(NW, NB1, B1)
    s2e, d2e = _pad_edges(src, dst, EPAD2)
    s2e = s2e.reshape(NW, NB2, B2)
    d2e = d2e.reshape(NW, NB2, B2)

    zer1 = jnp.zeros((B1, DIN), jnp.float32)
    zer2 = jnp.zeros((B2, DOUT), jnp.float32)
    one8 = jnp.ones((B1, DEGW), jnp.float32)
    zer8 = jnp.zeros((B1, DEGW), jnp.float32)

    aggp, degp = _seg128(features, s1, d1, zer1, one8, zer8)
    agg1 = aggp[:, :N]
    deg = degp[:, :N]

    z, s2 = _tc1(features, agg1, deg, W1_self.T, W1_neigh.T, b1[None],
                 W2_self.T, W2_neigh.T, b2[None])

    (agg2p,) = _seg64(z, s2e, d2e, zer2)
    out = _tc2(s2, agg2p[:, :N], deg)
    return out
